# baseline (speedup 1.0000x reference)
import jax
import jax.numpy as jnp
from jax.experimental import pallas as pl
from jax.experimental.pallas import tpu as pltpu

VOCAB, EMBED, HIDDEN = 100000, 256, 512
B, T = 16, 20
BT = B * T
VTILE = 6272
LOOKAHEAD = 2

_NT = (((1,), (1,)), ((), ()))


def _fused_body(idx_ref, e_any, enc_ref, wih0_ref, whh0_ref, bi0_ref,
                bh0_ref, wih1_ref, whh1_ref, bi1_ref, bh1_ref,
                w1_ref, b1_ref, w2_ref, b2_ref, et_ref, gb_ref, out_ref,
                emb_ref, ys_ref, proj_ref, wih0t_ref, whh0t_ref, wih1t_ref,
                whh1t_ref, sem):
    @pl.when(pl.program_id(0) == 0)
    def _():
        def row_copy(t, b):
            idx = idx_ref[b, t]
            return pltpu.make_async_copy(
                e_any.at[pl.ds(idx, 1), :],
                emb_ref.at[pl.ds(t * B + b, 1), :],
                sem)

        def issue_step(t):
            for b in range(B):
                row_copy(t, b).start()

        def wait_step(t):
            for b in range(B):
                row_copy(t, b).wait()

        for t in range(LOOKAHEAD):
            issue_step(t)

        wih0t_ref[...] = wih0_ref[...].T.astype(jnp.bfloat16)
        whh0t_ref[...] = whh0_ref[...].T.astype(jnp.bfloat16)
        wih1t_ref[...] = wih1_ref[...].T.astype(jnp.bfloat16)
        whh1t_ref[...] = whh1_ref[...].T.astype(jnp.bfloat16)

        def gru(x, h, wih, whh, bi, bh):
            gi = jnp.dot(x.astype(jnp.bfloat16), wih,
                         preferred_element_type=jnp.float32) + bi
            gh = jnp.dot(h.astype(jnp.bfloat16), whh,
                         preferred_element_type=jnp.float32) + bh
            i_r, i_z, i_n = gi[:, :HIDDEN], gi[:, HIDDEN:2 * HIDDEN], gi[:, 2 * HIDDEN:]
            h_r, h_z, h_n = gh[:, :HIDDEN], gh[:, HIDDEN:2 * HIDDEN], gh[:, 2 * HIDDEN:]
            r = jax.nn.sigmoid(i_r + h_r)
            z = jax.nn.sigmoid(i_z + h_z)
            n = jnp.tanh(i_n + r * h_n)
            return (1.0 - z) * n + z * h

        bi0 = bi0_ref[...].reshape(1, -1)
        bh0 = bh0_ref[...].reshape(1, -1)
        bi1 = bi1_ref[...].reshape(1, -1)
        bh1 = bh1_ref[...].reshape(1, -1)
        h0 = enc_ref[0]
        h1 = enc_ref[1]
        for t in range(T):
            if t + LOOKAHEAD < T:
                issue_step(t + LOOKAHEAD)
            wait_step(t)
            x = emb_ref[t * B:(t + 1) * B, :]
            h0new = gru(x, h0, wih0t_ref[...], whh0t_ref[...], bi0, bh0)
            if t > 0:
                h1 = gru(h0, h1, wih1t_ref[...], whh1t_ref[...], bi1, bh1)
                ys_ref[(t - 1) * B:t * B, :] = h1
            h0 = h0new
        h1 = gru(h0, h1, wih1t_ref[...], whh1t_ref[...], bi1, bh1)
        ys_ref[(T - 1) * B:T * B, :] = h1

        ys = ys_ref[...]
        hid = jnp.tanh(jax.lax.dot_general(ys, w1_ref[...], _NT,
                                           preferred_element_type=jnp.float32)
                       + b1_ref[...].reshape(1, -1))
        proj_ref[...] = jax.lax.dot_general(
            hid, w2_ref[...], _NT,
            preferred_element_type=jnp.float32) + b2_ref[...].reshape(1, -1)

    res = jax.lax.dot_general(
        proj_ref[...], et_ref[...], _NT,
        preferred_element_type=jnp.float32) + gb_ref[...]
    out_ref[...] = res.reshape(T, B, res.shape[-1])


def kernel(encoding, trg, E, W_ih0, W_hh0, b_ih0, b_hh0, W_ih1, W_hh1,
           b_ih1, b_hh1, W1, b1, W2, b2, g_b):
    idx = trg.astype(jnp.int32)
    nv = pl.cdiv(VOCAB, VTILE)

    vmem_full = pl.BlockSpec(memory_space=pltpu.MemorySpace.VMEM)
    logits_tb = pl.pallas_call(
        _fused_body,
        grid=(nv,),
        in_specs=[pl.BlockSpec(memory_space=pltpu.SMEM),
                  pl.BlockSpec(memory_space=pltpu.MemorySpace.HBM)]
                 + [vmem_full] * 13
                 + [pl.BlockSpec((VTILE, EMBED), lambda i: (i, 0)),
                    pl.BlockSpec((1, VTILE), lambda i: (0, i))],
        out_specs=pl.BlockSpec((T, B, VTILE), lambda i: (0, 0, i)),
        out_shape=jax.ShapeDtypeStruct((T, B, VOCAB), jnp.float32),
        scratch_shapes=[pltpu.VMEM((BT, EMBED), jnp.float32),
                        pltpu.VMEM((BT, HIDDEN), jnp.float32),
                        pltpu.VMEM((BT, EMBED), jnp.float32),
                        pltpu.VMEM((EMBED, 3 * HIDDEN), jnp.bfloat16),
                        pltpu.VMEM((HIDDEN, 3 * HIDDEN), jnp.bfloat16),
                        pltpu.VMEM((HIDDEN, 3 * HIDDEN), jnp.bfloat16),
                        pltpu.VMEM((HIDDEN, 3 * HIDDEN), jnp.bfloat16),
                        pltpu.SemaphoreType.DMA],
        compiler_params=pltpu.CompilerParams(
            dimension_semantics=("arbitrary",)),
    )(idx, E, encoding, W_ih0, W_hh0, b_ih0, b_hh0, W_ih1, W_hh1,
      b_ih1, b_hh1, W1, b1, W2, b2, E, g_b.reshape(1, VOCAB))

    return logits_tb.transpose(1, 0, 2)


# speedup vs baseline: 1.0122x; 1.0122x over previous
import jax
import jax.numpy as jnp
from jax.experimental import pallas as pl
from jax.experimental.pallas import tpu as pltpu

VOCAB, EMBED, HIDDEN = 100000, 256, 512
B, T = 16, 20
BT = B * T
VTILE = 8192
LOOKAHEAD = 2

_NT = (((1,), (1,)), ((), ()))


def _fused_body(idx_ref, e_any, enc_ref, wih0_ref, whh0_ref, bi0_ref,
                bh0_ref, wih1_ref, whh1_ref, bi1_ref, bh1_ref,
                w1_ref, b1_ref, w2_ref, b2_ref, et_ref, gb_ref, out_ref,
                emb_ref, ys_ref, proj_ref, wih0t_ref, whh0t_ref, wih1t_ref,
                whh1t_ref, sem):
    @pl.when(pl.program_id(0) == 0)
    def _():
        def row_copy(t, b):
            idx = idx_ref[b, t]
            return pltpu.make_async_copy(
                e_any.at[pl.ds(idx, 1), :],
                emb_ref.at[pl.ds(t * B + b, 1), :],
                sem)

        def issue_step(t):
            for b in range(B):
                row_copy(t, b).start()

        def wait_step(t):
            for b in range(B):
                row_copy(t, b).wait()

        for t in range(LOOKAHEAD):
            issue_step(t)

        wih0t_ref[...] = wih0_ref[...].T.astype(jnp.bfloat16)
        whh0t_ref[...] = whh0_ref[...].T.astype(jnp.bfloat16)
        wih1t_ref[...] = wih1_ref[...].T.astype(jnp.bfloat16)
        whh1t_ref[...] = whh1_ref[...].T.astype(jnp.bfloat16)

        def gru(x, h, wih, whh, bi, bh):
            gi = jnp.dot(x.astype(jnp.bfloat16), wih,
                         preferred_element_type=jnp.float32) + bi
            gh = jnp.dot(h.astype(jnp.bfloat16), whh,
                         preferred_element_type=jnp.float32) + bh
            i_r, i_z, i_n = gi[:, :HIDDEN], gi[:, HIDDEN:2 * HIDDEN], gi[:, 2 * HIDDEN:]
            h_r, h_z, h_n = gh[:, :HIDDEN], gh[:, HIDDEN:2 * HIDDEN], gh[:, 2 * HIDDEN:]
            r = jax.nn.sigmoid(i_r + h_r)
            z = jax.nn.sigmoid(i_z + h_z)
            n = jnp.tanh(i_n + r * h_n)
            return (1.0 - z) * n + z * h

        bi0 = bi0_ref[...].reshape(1, -1)
        bh0 = bh0_ref[...].reshape(1, -1)
        bi1 = bi1_ref[...].reshape(1, -1)
        bh1 = bh1_ref[...].reshape(1, -1)
        h0 = enc_ref[0]
        h1 = enc_ref[1]
        for t in range(T):
            if t + LOOKAHEAD < T:
                issue_step(t + LOOKAHEAD)
            wait_step(t)
            x = emb_ref[t * B:(t + 1) * B, :]
            h0new = gru(x, h0, wih0t_ref[...], whh0t_ref[...], bi0, bh0)
            if t > 0:
                h1 = gru(h0, h1, wih1t_ref[...], whh1t_ref[...], bi1, bh1)
                ys_ref[(t - 1) * B:t * B, :] = h1
            h0 = h0new
        h1 = gru(h0, h1, wih1t_ref[...], whh1t_ref[...], bi1, bh1)
        ys_ref[(T - 1) * B:T * B, :] = h1

        ys = ys_ref[...]
        hid = jnp.tanh(jax.lax.dot_general(ys, w1_ref[...], _NT,
                                           preferred_element_type=jnp.float32)
                       + b1_ref[...].reshape(1, -1))
        proj_ref[...] = jax.lax.dot_general(
            hid, w2_ref[...], _NT,
            preferred_element_type=jnp.float32) + b2_ref[...].reshape(1, -1)

    res = jax.lax.dot_general(
        proj_ref[...], et_ref[...], _NT,
        preferred_element_type=jnp.float32) + gb_ref[...]
    out_ref[...] = res.reshape(T, B, res.shape[-1])


def kernel(encoding, trg, E, W_ih0, W_hh0, b_ih0, b_hh0, W_ih1, W_hh1,
           b_ih1, b_hh1, W1, b1, W2, b2, g_b):
    idx = trg.astype(jnp.int32)
    nv = pl.cdiv(VOCAB, VTILE)

    vmem_full = pl.BlockSpec(memory_space=pltpu.MemorySpace.VMEM)
    logits_tb = pl.pallas_call(
        _fused_body,
        grid=(nv,),
        in_specs=[pl.BlockSpec(memory_space=pltpu.SMEM),
                  pl.BlockSpec(memory_space=pltpu.MemorySpace.HBM)]
                 + [vmem_full] * 13
                 + [pl.BlockSpec((VTILE, EMBED), lambda i: (i, 0)),
                    pl.BlockSpec((1, VTILE), lambda i: (0, i))],
        out_specs=pl.BlockSpec((T, B, VTILE), lambda i: (0, 0, i)),
        out_shape=jax.ShapeDtypeStruct((T, B, VOCAB), jnp.float32),
        scratch_shapes=[pltpu.VMEM((BT, EMBED), jnp.float32),
                        pltpu.VMEM((BT, HIDDEN), jnp.float32),
                        pltpu.VMEM((BT, EMBED), jnp.float32),
                        pltpu.VMEM((EMBED, 3 * HIDDEN), jnp.bfloat16),
                        pltpu.VMEM((HIDDEN, 3 * HIDDEN), jnp.bfloat16),
                        pltpu.VMEM((HIDDEN, 3 * HIDDEN), jnp.bfloat16),
                        pltpu.VMEM((HIDDEN, 3 * HIDDEN), jnp.bfloat16),
                        pltpu.SemaphoreType.DMA],
        compiler_params=pltpu.CompilerParams(
            dimension_semantics=("arbitrary",)),
    )(idx, E, encoding, W_ih0, W_hh0, b_ih0, b_hh0, W_ih1, W_hh1,
      b_ih1, b_hh1, W1, b1, W2, b2, E, g_b.reshape(1, VOCAB))

    return logits_tb.transpose(1, 0, 2)
